# baseline (speedup 1.0000x reference)
import functools
import math

import jax
import jax.numpy as jnp
from jax import lax
from jax.experimental import pallas as pl
from jax.experimental.pallas import tpu as pltpu
from jax.experimental.pallas import tpu_sc as plsc

_N_FIELDS = 26
_VOCAB = 100000
_EMB_DIM = 32
_D_MODEL = 1024
_BATCH = 4096
_SUM_EMB = _N_FIELDS * _EMB_DIM
_SCALE = math.sqrt(_EMB_DIM)

_NC = 2
_NS = 16
_NW = _NC * _NS
_L = 16
_BPW = _BATCH // _NW
_CPR = _VOCAB // 8
_FH = _N_FIELDS // 2
_HEMB = _FH * _EMB_DIM


def _gather_body(xt_hbm, tab_hbm, out_hbm, xall, idxv, offv, chunks, strip,
                 gsem, wsem):
    wid = lax.axis_index("s") * _NC + lax.axis_index("c")
    base = wid * _BPW
    pltpu.sync_copy(xt_hbm.at[:, pl.ds(base, _BPW)], xall)

    lanes = lax.iota(jnp.int32, _L)

    def field_body(f, carry):
        def build(e, c2):
            k = f * _EMB_DIM + e
            for g in range(_BPW // _L):
                xv = xall[f, pl.ds(g * _L, _L)]
                idxv[e, pl.ds(g * _L, _L)] = (
                    lax.shift_right_logical(xv, 3) + k * _CPR)
            return c2

        lax.fori_loop(0, _EMB_DIM, build, 0)
        for g in range(_BPW // _L):
            offv[g, :] = lax.bitwise_and(xall[f, pl.ds(g * _L, _L)], 7)

        for e in range(_EMB_DIM):
            pltpu.make_async_copy(
                tab_hbm.at[idxv.at[e]], chunks.at[e], gsem).start()
        for e in range(_EMB_DIM):
            pltpu.make_async_copy(
                tab_hbm.at[idxv.at[e]], chunks.at[e], gsem).wait()

        @pl.when(f > 0)
        def _():
            pltpu.make_async_copy(
                strip,
                out_hbm.at[pl.ds((f - 1) * _EMB_DIM, _EMB_DIM),
                           pl.ds(base, _BPW)],
                wsem,
            ).wait()

        def extract(e, c2):
            ev = jnp.zeros((_L,), jnp.int32) + e
            for g in range(_BPW // _L):
                b16 = g * _L + lanes
                v = plsc.load_gather(chunks, [ev, b16, offv[g, :]])
                strip[e, pl.ds(g * _L, _L)] = v
            return c2

        lax.fori_loop(0, _EMB_DIM, extract, 0)

        pltpu.make_async_copy(
            strip,
            out_hbm.at[pl.ds(f * _EMB_DIM, _EMB_DIM), pl.ds(base, _BPW)],
            wsem,
        ).start()
        return carry

    lax.fori_loop(0, _FH, field_body, 0)

    pltpu.make_async_copy(
        strip,
        out_hbm.at[pl.ds((_FH - 1) * _EMB_DIM, _EMB_DIM),
                   pl.ds(base, _BPW)],
        wsem,
    ).wait()


@functools.cache
def _make_gather():
    return pl.kernel(
        _gather_body,
        out_type=jax.ShapeDtypeStruct((_HEMB, _BATCH), jnp.float32),
        mesh=plsc.VectorSubcoreMesh(core_axis_name="c", subcore_axis_name="s"),
        scratch_types=[
            pltpu.VMEM((_FH, _BPW), jnp.int32),
            pltpu.VMEM((_EMB_DIM, _BPW), jnp.int32),
            pltpu.VMEM((_BPW // _L, _L), jnp.int32),
            pltpu.VMEM((_EMB_DIM, _BPW, 8), jnp.float32),
            pltpu.VMEM((_EMB_DIM, _BPW), jnp.float32),
            pltpu.SemaphoreType.DMA,
            pltpu.SemaphoreType.DMA,
        ],
        compiler_params=pltpu.CompilerParams(
            use_tc_tiling_on_sc=False, needs_layout_passes=False),
    )


def _proj_body(ea_ref, eb_ref, w_ref, b_ref, o_ref):
    dn = (((0,), (0,)), ((), ()))
    acc = jax.lax.dot_general(
        ea_ref[...], w_ref[pl.ds(0, _HEMB), :], dn,
        preferred_element_type=jnp.float32)
    acc += jax.lax.dot_general(
        eb_ref[...], w_ref[pl.ds(_HEMB, _HEMB), :], dn,
        preferred_element_type=jnp.float32)
    o_ref[...] = acc * _SCALE + b_ref[...]


_M_TILE = 512

_proj = pl.pallas_call(
    _proj_body,
    grid=(_BATCH // _M_TILE,),
    in_specs=[
        pl.BlockSpec((_HEMB, _M_TILE), lambda i: (0, i)),
        pl.BlockSpec((_HEMB, _M_TILE), lambda i: (0, i)),
        pl.BlockSpec((_SUM_EMB, _D_MODEL), lambda i: (0, 0)),
        pl.BlockSpec((1, _D_MODEL), lambda i: (0, 0)),
    ],
    out_specs=pl.BlockSpec((_M_TILE, _D_MODEL), lambda i: (i, 0)),
    out_shape=jax.ShapeDtypeStruct((_BATCH, _D_MODEL), jnp.float32),
)


def kernel(x, tables, W, b):
    xt = x.T
    gather = _make_gather()
    halves = []
    for h in range(2):
        tabh = tables[h * _FH:(h + 1) * _FH].transpose(0, 2, 1)
        tabc = tabh.reshape(_HEMB * _CPR, 8)
        halves.append(gather(xt[h * _FH:(h + 1) * _FH], tabc))
    return _proj(halves[0], halves[1], W, b.reshape(1, _D_MODEL))


# speedup vs baseline: 1.1216x; 1.1216x over previous
import functools
import math

import jax
import jax.numpy as jnp
from jax import lax
from jax.experimental import pallas as pl
from jax.experimental.pallas import tpu as pltpu
from jax.experimental.pallas import tpu_sc as plsc

_N_FIELDS = 26
_VOCAB = 100000
_EMB_DIM = 32
_D_MODEL = 1024
_BATCH = 4096
_SUM_EMB = _N_FIELDS * _EMB_DIM
_SCALE = math.sqrt(_EMB_DIM)

_NC = 2
_NS = 16
_NW = _NC * _NS
_L = 16
_BPW = _BATCH // _NW
_CPR = _VOCAB // 8


def _gather_body(xt_hbm, tab_hbm, out_hbm, xall, idxv, offv, chunks, strip,
                 gsem, wsem):
    wid = lax.axis_index("s") * _NC + lax.axis_index("c")
    base = wid * _BPW
    pltpu.sync_copy(xt_hbm.at[:, pl.ds(base, _BPW)], xall)

    lanes = lax.iota(jnp.int32, _L)

    def field_body(f, carry):
        def build(e, c2):
            k = f * _EMB_DIM + e
            for g in range(_BPW // _L):
                xv = xall[f, pl.ds(g * _L, _L)]
                idxv[e, pl.ds(g * _L, _L)] = (
                    lax.shift_right_logical(xv, 3) + k * _CPR)
            return c2

        lax.fori_loop(0, _EMB_DIM, build, 0)
        for g in range(_BPW // _L):
            offv[g, :] = lax.bitwise_and(xall[f, pl.ds(g * _L, _L)], 7)

        for e in range(_EMB_DIM):
            pltpu.make_async_copy(
                tab_hbm.at[idxv.at[e]], chunks.at[e], gsem).start()
        for e in range(_EMB_DIM):
            pltpu.make_async_copy(
                tab_hbm.at[idxv.at[e]], chunks.at[e], gsem).wait()

        @pl.when(f > 0)
        def _():
            pltpu.make_async_copy(
                strip,
                out_hbm.at[pl.ds((f - 1) * _EMB_DIM, _EMB_DIM),
                           pl.ds(base, _BPW)],
                wsem,
            ).wait()

        def extract(e, c2):
            ev = jnp.zeros((_L,), jnp.int32) + e
            for g in range(_BPW // _L):
                b16 = g * _L + lanes
                v = plsc.load_gather(chunks, [ev, b16, offv[g, :]])
                strip[e, pl.ds(g * _L, _L)] = v
            return c2

        lax.fori_loop(0, _EMB_DIM, extract, 0)

        pltpu.make_async_copy(
            strip,
            out_hbm.at[pl.ds(f * _EMB_DIM, _EMB_DIM), pl.ds(base, _BPW)],
            wsem,
        ).start()
        return carry

    lax.fori_loop(0, _N_FIELDS, field_body, 0)

    pltpu.make_async_copy(
        strip,
        out_hbm.at[pl.ds((_N_FIELDS - 1) * _EMB_DIM, _EMB_DIM),
                   pl.ds(base, _BPW)],
        wsem,
    ).wait()


@functools.cache
def _make_gather():
    return pl.kernel(
        _gather_body,
        out_type=jax.ShapeDtypeStruct((_SUM_EMB, _BATCH), jnp.float32),
        mesh=plsc.VectorSubcoreMesh(core_axis_name="c", subcore_axis_name="s"),
        scratch_types=[
            pltpu.VMEM((_N_FIELDS, _BPW), jnp.int32),
            pltpu.VMEM((_EMB_DIM, _BPW), jnp.int32),
            pltpu.VMEM((_BPW // _L, _L), jnp.int32),
            pltpu.VMEM((_EMB_DIM, _BPW, 8), jnp.float32),
            pltpu.VMEM((_EMB_DIM, _BPW), jnp.float32),
            pltpu.SemaphoreType.DMA,
            pltpu.SemaphoreType.DMA,
        ],
        compiler_params=pltpu.CompilerParams(
            use_tc_tiling_on_sc=False, needs_layout_passes=False),
    )


def _proj_body(e_ref, w_ref, b_ref, o_ref):
    acc = jax.lax.dot_general(
        e_ref[...], w_ref[...],
        dimension_numbers=(((0,), (0,)), ((), ())),
        preferred_element_type=jnp.float32)
    o_ref[...] = acc * _SCALE + b_ref[...]


_M_TILE = 512

_proj = pl.pallas_call(
    _proj_body,
    grid=(_BATCH // _M_TILE,),
    in_specs=[
        pl.BlockSpec((_SUM_EMB, _M_TILE), lambda i: (0, i)),
        pl.BlockSpec((_SUM_EMB, _D_MODEL), lambda i: (0, 0)),
        pl.BlockSpec((1, _D_MODEL), lambda i: (0, 0)),
    ],
    out_specs=pl.BlockSpec((_M_TILE, _D_MODEL), lambda i: (i, 0)),
    out_shape=jax.ShapeDtypeStruct((_BATCH, _D_MODEL), jnp.float32),
)


def kernel(x, tables, W, b):
    tabt = tables.transpose(0, 2, 1).reshape(_SUM_EMB, _VOCAB)
    tabc = tabt.reshape(_SUM_EMB * _CPR, 8)
    embt = _make_gather()(x.T, tabc)
    return _proj(embt, W, b.reshape(1, _D_MODEL))
